# Initial kernel scaffold; baseline (speedup 1.0000x reference)
import jax
import jax.numpy as jnp
from jax.experimental import pallas as pl


def kernel(edge_index_0, edge_index_1, edge_index_2, edge_index_3, edge_value_0, edge_value_1, edge_value_2, edge_value_3, num_nodes, weight):
    raise NotImplementedError("write your pallas kernel here")


# speedup vs baseline: 1.7702x; 1.7702x over previous
import functools

import jax
import jax.numpy as jnp
from jax import lax
from jax.experimental import pallas as pl

N_NODES = 100000
E_PER = 3200000
IN_CH = 4
OUT_CH = 2
TOTAL = IN_CH * E_PER

_BLK = 64000


def _weight_body(v_ref, f_ref, o_ref):
    v = v_ref[...]
    f = f_ref[...]
    o_ref[...] = v[None, :, :] * f[:, :, None]


def _weighted_values(vals, filt):
    grid = (E_PER // _BLK,)
    return pl.pallas_call(
        _weight_body,
        grid=grid,
        in_specs=[
            pl.BlockSpec((IN_CH, _BLK), lambda k: (jnp.int32(0), k)),
            pl.BlockSpec((OUT_CH, IN_CH), lambda k: (jnp.int32(0), jnp.int32(0))),
        ],
        out_specs=pl.BlockSpec((OUT_CH, IN_CH, _BLK),
                               lambda k: (jnp.int32(0), jnp.int32(0), k)),
        out_shape=jax.ShapeDtypeStruct((OUT_CH, IN_CH, E_PER), jnp.float32),
    )(vals, filt)


def kernel(edge_index_0, edge_index_1, edge_index_2, edge_index_3,
           edge_value_0, edge_value_1, edge_value_2, edge_value_3,
           num_nodes, weight):
    filt = jax.nn.softmax(weight, axis=1)
    idxs = [edge_index_0, edge_index_1, edge_index_2, edge_index_3]
    vals = jnp.stack([edge_value_0, edge_value_1, edge_value_2, edge_value_3])

    w = _weighted_values(vals, filt)
    w0 = w[0].reshape(TOTAL)
    w1 = w[1].reshape(TOTAL)

    ei = jnp.concatenate(idxs, axis=1)
    k = ei[0] * jnp.int64(num_nodes) + ei[1]

    ks, w0s, w1s = lax.sort((k, w0, w1), dimension=0, num_keys=1, is_stable=False)

    prev = jnp.concatenate([ks[:1] - 1, ks[:-1]])
    first = (ks != prev).astype(jnp.int32)
    seg = jnp.cumsum(first) - 1
    nseg = seg[-1] + 1

    val0 = jax.ops.segment_sum(w0s, seg, num_segments=TOTAL)
    val1 = jax.ops.segment_sum(w1s, seg, num_segments=TOTAL)

    uniq = jnp.zeros((TOTAL,), jnp.int64).at[seg].max(ks)
    fill = jnp.int64(num_nodes) * jnp.int64(num_nodes)
    pos = lax.iota(jnp.int32, TOTAL)
    uniq = jnp.where(pos < nseg, uniq, fill)

    row = uniq // jnp.int64(num_nodes)
    col = uniq - row * jnp.int64(num_nodes)
    idx = jnp.stack([row, col], axis=0)

    return (idx, val0, idx, val1)
